# Initial kernel scaffold; baseline (speedup 1.0000x reference)
import jax
import jax.numpy as jnp
from jax.experimental import pallas as pl


def kernel(user_emb, item_emb, edge_index, edge_weight):
    raise NotImplementedError("write your pallas kernel here")


# speedup vs baseline: 2.4802x; 2.4802x over previous
import functools

import jax
import jax.numpy as jnp
from jax import lax
from jax.experimental import pallas as pl
from jax.experimental.pallas import tpu as pltpu
from jax.experimental.pallas import tpu_sc as plsc

USERS = 15000
ITEMS = 35000
NNODES = USERS + ITEMS
EMB = 64
NEDGES = 800000

NCORES = 2
NSUB = 16
HALF = 25088
TROWS = HALF // NSUB
NPAD = NCORES * HALF
DUMMY = HALF
ACC_ROWS = HALF + 8

EPAD = 819200
SUB = 128
CHUNK = 256
NSUBCH = CHUNK // SUB
EROWS = EPAD // SUB
WROWS = EROWS // NSUB
NROUNDS = EPAD // NSUB // CHUNK

_MESH = plsc.VectorSubcoreMesh(
    core_axis_name="c", subcore_axis_name="s",
    num_cores=NCORES, num_subcores=NSUB)


@functools.partial(
    pl.kernel,
    out_type=jax.ShapeDtypeStruct((NPAD, EMB), jnp.float32),
    mesh=_MESH,
    scratch_types=[
        pltpu.VMEM((NSUBCH, SUB), jnp.int32),
        pltpu.VMEM((NSUBCH, SUB), jnp.int32),
        pltpu.VMEM((NSUBCH, SUB), jnp.float32),
        pltpu.VMEM((CHUNK, EMB), jnp.float32),
        pltpu.VMEM_SHARED((ACC_ROWS, EMB), jnp.float32),
        pltpu.SemaphoreType.DMA,
    ],
    compiler_params=pltpu.CompilerParams(use_tc_tiling_on_sc=False),
)
def _layer(ego, srcr, dstr, wgtr, out, sidx, didx, wbuf, rows, acc, sem):
    c = lax.axis_index("c")
    s = lax.axis_index("s")
    base_c = c * HALF
    rbase = s * TROWS

    def _zrow(i, carry):
        for k in range(EMB // 16):
            rows[i, pl.ds(k * 16, 16)] = jnp.zeros((16,), jnp.float32)
        return carry
    lax.fori_loop(0, CHUNK, _zrow, 0)
    for b in range(TROWS // CHUNK):
        pltpu.sync_copy(rows.at[pl.ds(0, CHUNK)],
                        acc.at[pl.ds(rbase + b * CHUNK, CHUNK)])
    rem = TROWS % CHUNK
    if rem:
        pltpu.sync_copy(rows.at[pl.ds(0, rem)],
                        acc.at[pl.ds(rbase + TROWS - rem, rem)])
    plsc.subcore_barrier()

    def _round(r, carry):
        rowb = s * WROWS + r * NSUBCH
        pltpu.sync_copy(srcr.at[pl.ds(rowb, NSUBCH)], sidx)
        pltpu.sync_copy(dstr.at[pl.ds(rowb, NSUBCH)], didx)
        pltpu.sync_copy(wgtr.at[pl.ds(rowb, NSUBCH)], wbuf)
        cps = [pltpu.async_copy(ego.at[sidx.at[j]],
                                rows.at[pl.ds(j * SUB, SUB)], sem)
               for j in range(NSUBCH)]
        for j in range(NSUBCH):
            cps[j].wait()
            for k in range(SUB // 16):
                v = didx[j, pl.ds(k * 16, 16)]
                ok = (v >= base_c) & (v < base_c + HALF)
                didx[j, pl.ds(k * 16, 16)] = jnp.where(ok, v - base_c, DUMMY)
            def _scale(g, carry, j=j):
                wv = wbuf[j, pl.ds(g * 16, 16)]
                for i in range(16):
                    e = j * SUB + g * 16 + i
                    w = wv[i]
                    for k in range(EMB // 16):
                        rows[e, pl.ds(k * 16, 16)] = (
                            rows[e, pl.ds(k * 16, 16)] * w)
                return carry
            lax.fori_loop(0, SUB // 16, _scale, 0)
            pltpu.sync_copy(rows.at[pl.ds(j * SUB, SUB)],
                            acc.at[didx.at[j]], add=True)
        return carry
    lax.fori_loop(0, NROUNDS, _round, 0)
    plsc.subcore_barrier()

    for b in range(TROWS // CHUNK):
        pltpu.sync_copy(acc.at[pl.ds(rbase + b * CHUNK, CHUNK)], rows)
        pltpu.sync_copy(rows, out.at[pl.ds(base_c + rbase + b * CHUNK, CHUNK)])
    if rem:
        pltpu.sync_copy(acc.at[pl.ds(rbase + TROWS - rem, rem)],
                        rows.at[pl.ds(0, rem)])
        pltpu.sync_copy(rows.at[pl.ds(0, rem)],
                        out.at[pl.ds(base_c + rbase + TROWS - rem, rem)])


def _mean_body(a_ref, b_ref, c_ref, o_ref):
    o_ref[...] = (a_ref[...] + b_ref[...] + c_ref[...]) * (1.0 / 3.0)


_MEAN_BLOCK = 2000


def _mean3(e1, e2, e3):
    spec = pl.BlockSpec((_MEAN_BLOCK, EMB), lambda i: (i, 0))
    return pl.pallas_call(
        _mean_body,
        grid=(NNODES // _MEAN_BLOCK,),
        in_specs=[spec, spec, spec],
        out_specs=spec,
        out_shape=jax.ShapeDtypeStruct((NNODES, EMB), jnp.float32),
    )(e1, e2, e3)


def kernel(user_emb, item_emb, edge_index, edge_weight):
    ego0 = jnp.concatenate([user_emb, item_emb], axis=0)
    ego0 = jnp.pad(ego0, ((0, NPAD - NNODES), (0, 0)))
    dst = edge_index[0].astype(jnp.int32)
    src = edge_index[1].astype(jnp.int32)
    w = edge_weight.astype(jnp.float32)
    src = jnp.pad(src, (0, EPAD - NEDGES)).reshape(EROWS, SUB)
    dst = jnp.pad(dst, (0, EPAD - NEDGES),
                  constant_values=jnp.int32(2 ** 30)).reshape(EROWS, SUB)
    w = jnp.pad(w, (0, EPAD - NEDGES)).reshape(EROWS, SUB)

    e1 = _layer(ego0, src, dst, w)
    e2 = _layer(e1, src, dst, w)
    e3 = _layer(e2, src, dst, w)
    fin = _mean3(e1, e2, e3)
    return fin[:USERS], fin[USERS:]


# speedup vs baseline: 2.9187x; 1.1768x over previous
import functools

import jax
import jax.numpy as jnp
from jax import lax
from jax.experimental import pallas as pl
from jax.experimental.pallas import tpu as pltpu
from jax.experimental.pallas import tpu_sc as plsc

USERS = 15000
ITEMS = 35000
NNODES = USERS + ITEMS
EMB = 64
NEDGES = 800000

NCORES = 2
NSUB = 16
HALF = 25088
TROWS = HALF // NSUB
NPAD = NCORES * HALF
DUMMY = HALF
ACC_ROWS = HALF + 8

EPAD = 819200
SUB = 128
CHUNK = 256
NSUBCH = CHUNK // SUB
EROWS = EPAD // SUB
WROWS = EROWS // NSUB
NROUNDS = EPAD // NSUB // CHUNK
NPAIRS = NROUNDS // 2

_MESH = plsc.VectorSubcoreMesh(
    core_axis_name="c", subcore_axis_name="s",
    num_cores=NCORES, num_subcores=NSUB)


@functools.partial(
    pl.kernel,
    out_type=jax.ShapeDtypeStruct((NPAD, EMB), jnp.float32),
    mesh=_MESH,
    scratch_types=[
        pltpu.VMEM((NSUBCH, 3, SUB), jnp.int32),
        pltpu.VMEM((NSUBCH, 3, SUB), jnp.int32),
        pltpu.VMEM((CHUNK, EMB), jnp.float32),
        pltpu.VMEM_SHARED((ACC_ROWS, EMB), jnp.float32),
        pltpu.SemaphoreType.DMA,
        pltpu.SemaphoreType.DMA,
        pltpu.SemaphoreType.DMA,
        pltpu.SemaphoreType.DMA,
        pltpu.SemaphoreType.DMA,
        pltpu.SemaphoreType.DMA,
    ],
    compiler_params=pltpu.CompilerParams(
        use_tc_tiling_on_sc=False, needs_layout_passes=False),
)
def _layer(ego, edges, out, ebA, ebB, rows, acc,
           semA, semB, semg0, semg1, sems0, sems1):
    c = lax.axis_index("c")
    s = lax.axis_index("s")
    base_c = c * HALF
    rbase = s * TROWS
    ebase = s * WROWS
    semg = (semg0, semg1)
    sems = (sems0, sems1)

    def _zrow(i, carry):
        for k in range(EMB // 16):
            rows[i, pl.ds(k * 16, 16)] = jnp.zeros((16,), jnp.float32)
        return carry
    lax.fori_loop(0, CHUNK, _zrow, 0)
    for b in range(TROWS // CHUNK):
        pltpu.sync_copy(rows.at[pl.ds(0, CHUNK)],
                        acc.at[pl.ds(rbase + b * CHUNK, CHUNK)])
    rem = TROWS % CHUNK
    if rem:
        pltpu.sync_copy(rows.at[pl.ds(0, rem)],
                        acc.at[pl.ds(rbase + TROWS - rem, rem)])
    plsc.subcore_barrier()

    pltpu.async_copy(edges.at[pl.ds(ebase, NSUBCH)], ebA, semA)
    pltpu.async_copy(edges.at[pl.ds(ebase + NSUBCH, NSUBCH)], ebB, semB)

    def _round(eb, sem, rowb, prefetch_rowb, do_prefetch):
        pltpu.make_async_copy(edges.at[pl.ds(rowb, NSUBCH)], eb, sem).wait()
        gcp = [pltpu.async_copy(ego.at[eb.at[j, 0]],
                                rows.at[pl.ds(j * SUB, SUB)], semg[j])
               for j in range(NSUBCH)]
        scp = []
        for j in range(NSUBCH):
            gcp[j].wait()
            for k in range(SUB // 16):
                v = eb[j, 1, pl.ds(k * 16, 16)]
                ok = (v >= base_c) & (v < base_c + HALF)
                eb[j, 1, pl.ds(k * 16, 16)] = jnp.where(ok, v - base_c, DUMMY)

            @plsc.parallel_loop(0, SUB // 16, 1, unroll=2)
            def _scale(g, j=j):
                wv = plsc.bitcast(eb[j, 2, pl.ds(g * 16, 16)], jnp.float32)
                for i in range(16):
                    e = j * SUB + g * 16 + i
                    w = wv[i]
                    for k in range(EMB // 16):
                        rows[e, pl.ds(k * 16, 16)] = (
                            rows[e, pl.ds(k * 16, 16)] * w)

            scp.append(pltpu.async_copy(rows.at[pl.ds(j * SUB, SUB)],
                                        acc.at[eb.at[j, 1]], sems[j],
                                        add=True))
        for cp in scp:
            cp.wait()

        @pl.when(do_prefetch)
        def _():
            pltpu.async_copy(edges.at[pl.ds(prefetch_rowb, NSUBCH)], eb, sem)

    def _pair(i, carry):
        rowbA = ebase + 2 * i * NSUBCH
        rowbB = rowbA + NSUBCH
        _round(ebA, semA, rowbA, rowbA + 2 * NSUBCH, i + 1 < NPAIRS)
        _round(ebB, semB, rowbB, rowbB + 2 * NSUBCH, i + 1 < NPAIRS)
        return carry
    lax.fori_loop(0, NPAIRS, _pair, 0)
    plsc.subcore_barrier()

    for b in range(TROWS // CHUNK):
        pltpu.sync_copy(acc.at[pl.ds(rbase + b * CHUNK, CHUNK)], rows)
        pltpu.sync_copy(rows, out.at[pl.ds(base_c + rbase + b * CHUNK, CHUNK)])
    if rem:
        pltpu.sync_copy(acc.at[pl.ds(rbase + TROWS - rem, rem)],
                        rows.at[pl.ds(0, rem)])
        pltpu.sync_copy(rows.at[pl.ds(0, rem)],
                        out.at[pl.ds(base_c + rbase + TROWS - rem, rem)])


def _mean_body(a_ref, b_ref, c_ref, o_ref):
    o_ref[...] = (a_ref[...] + b_ref[...] + c_ref[...]) * (1.0 / 3.0)


_MEAN_BLOCK = 2000


def _mean3(e1, e2, e3):
    spec = pl.BlockSpec((_MEAN_BLOCK, EMB), lambda i: (i, 0))
    return pl.pallas_call(
        _mean_body,
        grid=(NNODES // _MEAN_BLOCK,),
        in_specs=[spec, spec, spec],
        out_specs=spec,
        out_shape=jax.ShapeDtypeStruct((NNODES, EMB), jnp.float32),
    )(e1, e2, e3)


def kernel(user_emb, item_emb, edge_index, edge_weight):
    ego0 = jnp.concatenate([user_emb, item_emb], axis=0)
    ego0 = jnp.pad(ego0, ((0, NPAD - NNODES), (0, 0)))
    dst = edge_index[0].astype(jnp.int32)
    src = edge_index[1].astype(jnp.int32)
    w = edge_weight.astype(jnp.float32)
    src = jnp.pad(src, (0, EPAD - NEDGES))
    dst = jnp.pad(dst, (0, EPAD - NEDGES), constant_values=jnp.int32(2 ** 30))
    wbits = lax.bitcast_convert_type(jnp.pad(w, (0, EPAD - NEDGES)), jnp.int32)
    edges = jnp.stack([src.reshape(EROWS, SUB), dst.reshape(EROWS, SUB),
                       wbits.reshape(EROWS, SUB)], axis=1)

    e1 = _layer(ego0, edges)
    e2 = _layer(e1, edges)
    e3 = _layer(e2, edges)
    fin = _mean3(e1, e2, e3)
    return fin[:USERS], fin[USERS:]


# speedup vs baseline: 5.5315x; 1.8952x over previous
import functools

import jax
import jax.numpy as jnp
from jax import lax
from jax.experimental import pallas as pl
from jax.experimental.pallas import tpu as pltpu
from jax.experimental.pallas import tpu_sc as plsc

USERS = 15000
ITEMS = 35000
NNODES = USERS + ITEMS
EMB = 64
NEDGES = 800000

NCORES = 2
NSUB = 16
HALF = 25088
TROWS = HALF // NSUB
NPAD = NCORES * HALF
DUMMY = HALF
ACC_ROWS = HALF + 8

EPAD = 819200
SUB = 128
EROWS = EPAD // SUB
WROWS = EROWS // NSUB

PADR = 3328
LTROWS = PADR // NSUB
NSUBCH = 2
NROUNDS = LTROWS // NSUBCH
NPAIRS = NROUNDS // 2

BROWS = 8
NBLK = WROWS // BROWS
FLUSH = BROWS * SUB

_MESH = plsc.VectorSubcoreMesh(
    core_axis_name="c", subcore_axis_name="s",
    num_cores=NCORES, num_subcores=NSUB)

_SC_PARAMS = pltpu.CompilerParams(
    use_tc_tiling_on_sc=False, needs_layout_passes=False)


@functools.partial(
    pl.kernel,
    out_type=(
        jax.ShapeDtypeStruct((NCORES, PADR, SUB), jnp.int32),
        jax.ShapeDtypeStruct((NCORES, PADR, SUB), jnp.int32),
        jax.ShapeDtypeStruct((NCORES, PADR, SUB), jnp.int32),
        jax.ShapeDtypeStruct((NCORES, 16), jnp.int32),
    ),
    mesh=_MESH,
    scratch_types=[
        pltpu.VMEM((BROWS, 3, SUB), jnp.int32),
        pltpu.VMEM((BROWS, 3, SUB), jnp.int32),
        pltpu.VMEM((16, SUB), jnp.int32),
        pltpu.VMEM((16, SUB), jnp.int32),
        pltpu.VMEM((16, SUB), jnp.int32),
        pltpu.VMEM((16,), jnp.int32),
        pltpu.SMEM((8,), jnp.int32),
        pltpu.SemaphoreType.DMA,
        pltpu.SemaphoreType.DMA,
    ],
    compiler_params=_SC_PARAMS,
)
def _partition(edges, srcp, dstp, wp, counts,
               ebA, ebB, csrc, cdst, cw, cbuf, cnt, semA, semB):
    c = lax.axis_index("c")
    s = lax.axis_index("s")
    base_c = c * HALF
    ibase = s * WROWS

    @pl.when(s == 0)
    def _():
        cnt[0] = jnp.int32(0)
    plsc.subcore_barrier()

    pltpu.async_copy(edges.at[pl.ds(ibase, BROWS)], ebA, semA)
    pltpu.async_copy(edges.at[pl.ds(ibase + BROWS, BROWS)], ebB, semB)

    def _flush(lp):
        flushed = lp >= FLUSH

        @pl.when(flushed)
        def _():
            grow = plsc.fetch_and_add(cnt.at[0], jnp.int32(BROWS),
                                      subcore_id=0)

            @pl.when(grow <= PADR - BROWS)
            def _():
                pltpu.sync_copy(csrc.at[pl.ds(0, BROWS)],
                                srcp.at[c].at[pl.ds(grow, BROWS)])
                pltpu.sync_copy(cdst.at[pl.ds(0, BROWS)],
                                dstp.at[c].at[pl.ds(grow, BROWS)])
                pltpu.sync_copy(cw.at[pl.ds(0, BROWS)],
                                wp.at[c].at[pl.ds(grow, BROWS)])
            for r in range(BROWS):
                for g in range(SUB // 16):
                    sl = pl.ds(g * 16, 16)
                    csrc[r, sl] = csrc[BROWS + r, sl]
                    cdst[r, sl] = cdst[BROWS + r, sl]
                    cw[r, sl] = cw[BROWS + r, sl]
        return jnp.where(flushed, lp - FLUSH, lp)

    def _block(eb, sem, rowb, pre_rowb, do_pre, lp):
        pltpu.make_async_copy(edges.at[pl.ds(rowb, BROWS)], eb, sem).wait()
        for r in range(BROWS):
            for g in range(SUB // 16):
                sl = pl.ds(g * 16, 16)
                sv = eb[r, 0, sl]
                dv = eb[r, 1, sl]
                wv = eb[r, 2, sl]
                ok = (dv >= base_c) & (dv < base_c + HALF)
                cs = plsc.cumsum(jnp.where(ok, jnp.int32(1), jnp.int32(0)))
                pos = jnp.maximum(lp + cs - 1, 0)
                row = jax.lax.shift_right_logical(pos, 7)
                col = jax.lax.bitwise_and(pos, 127)
                plsc.store_scatter(csrc, [row, col], sv, mask=ok)
                plsc.store_scatter(cdst, [row, col], dv, mask=ok)
                plsc.store_scatter(cw, [row, col], wv, mask=ok)
                lp = lp + cs[15]

        @pl.when(do_pre)
        def _():
            pltpu.async_copy(edges.at[pl.ds(pre_rowb, BROWS)], eb, sem)
        return _flush(lp)

    def _pair(i, lp):
        rowbA = ibase + 2 * i * BROWS
        rowbB = rowbA + BROWS
        lp = _block(ebA, semA, rowbA, rowbA + 2 * BROWS, i + 1 < NBLK // 2, lp)
        lp = _block(ebB, semB, rowbB, rowbB + 2 * BROWS, i + 1 < NBLK // 2, lp)
        return lp
    lp = lax.fori_loop(0, NBLK // 2, _pair, jnp.int32(0))

    for r in range(BROWS):
        for g in range(SUB // 16):
            sl = pl.ds(g * 16, 16)
            pvec = lax.iota(jnp.int32, 16) + (r * SUB + g * 16)
            keep = pvec < lp
            zero = jnp.zeros((16,), jnp.int32)
            csrc[r, sl] = jnp.where(keep, csrc[r, sl], zero)
            cdst[r, sl] = jnp.where(keep, cdst[r, sl], zero)
            cw[r, sl] = jnp.where(keep, cw[r, sl], zero)

    @pl.when(lp > 0)
    def _():
        grow = plsc.fetch_and_add(cnt.at[0], jnp.int32(BROWS), subcore_id=0)

        @pl.when(grow <= PADR - BROWS)
        def _():
            pltpu.sync_copy(csrc.at[pl.ds(0, BROWS)],
                            srcp.at[c].at[pl.ds(grow, BROWS)])
            pltpu.sync_copy(cdst.at[pl.ds(0, BROWS)],
                            dstp.at[c].at[pl.ds(grow, BROWS)])
            pltpu.sync_copy(cw.at[pl.ds(0, BROWS)],
                            wp.at[c].at[pl.ds(grow, BROWS)])
    plsc.subcore_barrier()

    @pl.when(s == 0)
    def _():
        total = cnt[0]
        cbuf[pl.ds(0, 16)] = jnp.full((16,), total, jnp.int32)
        pltpu.sync_copy(cbuf, counts.at[c])


@functools.partial(
    pl.kernel,
    out_type=jax.ShapeDtypeStruct((NPAD, EMB), jnp.float32),
    mesh=_MESH,
    scratch_types=[
        pltpu.VMEM((NSUBCH, SUB), jnp.int32),
        pltpu.VMEM((NSUBCH, SUB), jnp.int32),
        pltpu.VMEM((NSUBCH, SUB), jnp.int32),
        pltpu.VMEM((NSUBCH, SUB), jnp.int32),
        pltpu.VMEM((NSUBCH, SUB), jnp.int32),
        pltpu.VMEM((NSUBCH, SUB), jnp.int32),
        pltpu.VMEM((NSUBCH * SUB, EMB), jnp.float32),
        pltpu.VMEM((16,), jnp.int32),
        pltpu.VMEM_SHARED((ACC_ROWS, EMB), jnp.float32),
        pltpu.SemaphoreType.DMA,
        pltpu.SemaphoreType.DMA,
        pltpu.SemaphoreType.DMA,
        pltpu.SemaphoreType.DMA,
        pltpu.SemaphoreType.DMA,
        pltpu.SemaphoreType.DMA,
    ],
    compiler_params=_SC_PARAMS,
)
def _layer(ego, srcp, dstp, wp, counts, out,
           sbA, sbB, dbA, dbB, wbA, wbB, rows, cbuf, acc,
           semA, semB, semg0, semg1, sems0, sems1):
    c = lax.axis_index("c")
    s = lax.axis_index("s")
    base_c = c * HALF
    rbase = s * TROWS
    lbase = s * LTROWS
    semg = (semg0, semg1)
    sems = (sems0, sems1)

    pltpu.sync_copy(counts.at[c], cbuf)
    cnt_rows = cbuf[pl.ds(0, 16)][0]

    CH = NSUBCH * SUB

    def _zrow(i, carry):
        for k in range(EMB // 16):
            rows[i, pl.ds(k * 16, 16)] = jnp.zeros((16,), jnp.float32)
        return carry
    lax.fori_loop(0, CH, _zrow, 0)
    for b in range(TROWS // CH):
        pltpu.sync_copy(rows.at[pl.ds(0, CH)],
                        acc.at[pl.ds(rbase + b * CH, CH)])
    rem = TROWS % CH
    if rem:
        pltpu.sync_copy(rows.at[pl.ds(0, rem)],
                        acc.at[pl.ds(rbase + TROWS - rem, rem)])
    plsc.subcore_barrier()

    def _stage(bufs, sem, rowb):
        sb, db, wb = bufs
        pltpu.async_copy(srcp.at[c].at[pl.ds(rowb, NSUBCH)], sb, sem)
        pltpu.async_copy(dstp.at[c].at[pl.ds(rowb, NSUBCH)], db, sem)
        pltpu.async_copy(wp.at[c].at[pl.ds(rowb, NSUBCH)], wb, sem)

    def _wait_stage(bufs, sem, rowb):
        sb, db, wb = bufs
        pltpu.make_async_copy(srcp.at[c].at[pl.ds(rowb, NSUBCH)], sb,
                              sem).wait()
        pltpu.make_async_copy(dstp.at[c].at[pl.ds(rowb, NSUBCH)], db,
                              sem).wait()
        pltpu.make_async_copy(wp.at[c].at[pl.ds(rowb, NSUBCH)], wb,
                              sem).wait()

    bufsA = (sbA, dbA, wbA)
    bufsB = (sbB, dbB, wbB)

    @pl.when(lbase < cnt_rows)
    def _():
        _stage(bufsA, semA, lbase)

    @pl.when(lbase + NSUBCH < cnt_rows)
    def _():
        _stage(bufsB, semB, lbase + NSUBCH)

    def _round(bufs, sem, rowb, pre_rowb, may_prefetch):
        sb, db, wb = bufs
        active = rowb < cnt_rows

        @pl.when(active)
        def _():
            _wait_stage(bufs, sem, rowb)
            gcp = [pltpu.async_copy(ego.at[sb.at[j]],
                                    rows.at[pl.ds(j * SUB, SUB)], semg[j])
                   for j in range(NSUBCH)]
            scp = []
            for j in range(NSUBCH):
                gcp[j].wait()
                for k in range(SUB // 16):
                    v = db[j, pl.ds(k * 16, 16)]
                    ok = (v >= base_c) & (v < base_c + HALF)
                    db[j, pl.ds(k * 16, 16)] = jnp.where(ok, v - base_c,
                                                         DUMMY)

                @plsc.parallel_loop(0, SUB // 16, 1, unroll=2)
                def _scale(g, j=j):
                    wv = plsc.bitcast(wb[j, pl.ds(g * 16, 16)], jnp.float32)
                    for i in range(16):
                        e = j * SUB + g * 16 + i
                        w = wv[i]
                        for k in range(EMB // 16):
                            rows[e, pl.ds(k * 16, 16)] = (
                                rows[e, pl.ds(k * 16, 16)] * w)

                scp.append(pltpu.async_copy(rows.at[pl.ds(j * SUB, SUB)],
                                            acc.at[db.at[j]], sems[j],
                                            add=True))
            for cp in scp:
                cp.wait()

            @pl.when(may_prefetch & (pre_rowb < cnt_rows))
            def _():
                _stage(bufs, sem, pre_rowb)

    def _pair(i, carry):
        rowbA = lbase + 2 * i * NSUBCH
        rowbB = rowbA + NSUBCH
        may = i + 1 < NPAIRS
        _round(bufsA, semA, rowbA, rowbA + 2 * NSUBCH, may)
        _round(bufsB, semB, rowbB, rowbB + 2 * NSUBCH, may)
        return carry
    lax.fori_loop(0, NPAIRS, _pair, 0)
    plsc.subcore_barrier()

    for b in range(TROWS // CH):
        pltpu.sync_copy(acc.at[pl.ds(rbase + b * CH, CH)], rows)
        pltpu.sync_copy(rows, out.at[pl.ds(base_c + rbase + b * CH, CH)])
    if rem:
        pltpu.sync_copy(acc.at[pl.ds(rbase + TROWS - rem, rem)],
                        rows.at[pl.ds(0, rem)])
        pltpu.sync_copy(rows.at[pl.ds(0, rem)],
                        out.at[pl.ds(base_c + rbase + TROWS - rem, rem)])


def _mean_body(a_ref, b_ref, c_ref, o_ref):
    o_ref[...] = (a_ref[...] + b_ref[...] + c_ref[...]) * (1.0 / 3.0)


_MEAN_BLOCK = 2000


def _mean3(e1, e2, e3):
    spec = pl.BlockSpec((_MEAN_BLOCK, EMB), lambda i: (i, 0))
    return pl.pallas_call(
        _mean_body,
        grid=(NNODES // _MEAN_BLOCK,),
        in_specs=[spec, spec, spec],
        out_specs=spec,
        out_shape=jax.ShapeDtypeStruct((NNODES, EMB), jnp.float32),
    )(e1, e2, e3)


def kernel(user_emb, item_emb, edge_index, edge_weight):
    ego0 = jnp.concatenate([user_emb, item_emb], axis=0)
    ego0 = jnp.pad(ego0, ((0, NPAD - NNODES), (0, 0)))
    dst = edge_index[0].astype(jnp.int32)
    src = edge_index[1].astype(jnp.int32)
    w = edge_weight.astype(jnp.float32)
    src = jnp.pad(src, (0, EPAD - NEDGES))
    dst = jnp.pad(dst, (0, EPAD - NEDGES), constant_values=jnp.int32(2 ** 30))
    wbits = lax.bitcast_convert_type(jnp.pad(w, (0, EPAD - NEDGES)), jnp.int32)
    edges = jnp.stack([src.reshape(EROWS, SUB), dst.reshape(EROWS, SUB),
                       wbits.reshape(EROWS, SUB)], axis=1)

    srcp, dstp, wp, counts = _partition(edges)
    e1 = _layer(ego0, srcp, dstp, wp, counts)
    e2 = _layer(e1, srcp, dstp, wp, counts)
    e3 = _layer(e2, srcp, dstp, wp, counts)
    fin = _mean3(e1, e2, e3)
    return fin[:USERS], fin[USERS:]


# speedup vs baseline: 6.1144x; 1.1054x over previous
import functools

import jax
import jax.numpy as jnp
from jax import lax
from jax.experimental import pallas as pl
from jax.experimental.pallas import tpu as pltpu
from jax.experimental.pallas import tpu_sc as plsc

USERS = 15000
ITEMS = 35000
NNODES = USERS + ITEMS
EMB = 64
NEDGES = 800000

NCORES = 2
NSUB = 16
HALF = 25088
TROWS = HALF // NSUB
NPAD = NCORES * HALF
DUMMY = HALF
ACC_ROWS = HALF + 8

EPAD = 819200
SUB = 128
EROWS = EPAD // SUB
WROWS = EROWS // NSUB

PADR = 3456
LTROWS = PADR // NSUB
UPB = 24
NBODY = LTROWS // UPB
SGROUPS = LTROWS // 4

BROWS = 8
NBLK = WROWS // BROWS
FLUSH = BROWS * SUB

_MESH = plsc.VectorSubcoreMesh(
    core_axis_name="c", subcore_axis_name="s",
    num_cores=NCORES, num_subcores=NSUB)

_SC_PARAMS = pltpu.CompilerParams(
    use_tc_tiling_on_sc=False, needs_layout_passes=False)


@functools.partial(
    pl.kernel,
    out_type=(
        jax.ShapeDtypeStruct((NCORES, PADR, SUB), jnp.int32),
        jax.ShapeDtypeStruct((NCORES, PADR, SUB), jnp.int32),
        jax.ShapeDtypeStruct((NCORES, PADR, SUB), jnp.int32),
        jax.ShapeDtypeStruct((NCORES, 16), jnp.int32),
    ),
    mesh=_MESH,
    scratch_types=[
        pltpu.VMEM((BROWS, 3, SUB), jnp.int32),
        pltpu.VMEM((BROWS, 3, SUB), jnp.int32),
        pltpu.VMEM((16, SUB), jnp.int32),
        pltpu.VMEM((16, SUB), jnp.int32),
        pltpu.VMEM((16, SUB), jnp.int32),
        pltpu.VMEM((16,), jnp.int32),
        pltpu.SMEM((8,), jnp.int32),
        pltpu.SemaphoreType.DMA,
        pltpu.SemaphoreType.DMA,
    ],
    compiler_params=_SC_PARAMS,
)
def _partition(edges, srcp, dstp, wp, counts,
               ebA, ebB, csrc, cdst, cw, cbuf, cnt, semA, semB):
    c = lax.axis_index("c")
    s = lax.axis_index("s")
    base_c = c * HALF
    ibase = s * WROWS

    @pl.when(s == 0)
    def _():
        cnt[0] = jnp.int32(0)
    plsc.subcore_barrier()

    pltpu.async_copy(edges.at[pl.ds(ibase, BROWS)], ebA, semA)
    pltpu.async_copy(edges.at[pl.ds(ibase + BROWS, BROWS)], ebB, semB)

    def _flush(lp):
        flushed = lp >= FLUSH

        @pl.when(flushed)
        def _():
            grow = plsc.fetch_and_add(cnt.at[0], jnp.int32(BROWS),
                                      subcore_id=0)

            @pl.when(grow <= PADR - BROWS)
            def _():
                pltpu.sync_copy(csrc.at[pl.ds(0, BROWS)],
                                srcp.at[c].at[pl.ds(grow, BROWS)])
                pltpu.sync_copy(cdst.at[pl.ds(0, BROWS)],
                                dstp.at[c].at[pl.ds(grow, BROWS)])
                pltpu.sync_copy(cw.at[pl.ds(0, BROWS)],
                                wp.at[c].at[pl.ds(grow, BROWS)])
            for r in range(BROWS):
                for g in range(SUB // 16):
                    sl = pl.ds(g * 16, 16)
                    csrc[r, sl] = csrc[BROWS + r, sl]
                    cdst[r, sl] = cdst[BROWS + r, sl]
                    cw[r, sl] = cw[BROWS + r, sl]
        return jnp.where(flushed, lp - FLUSH, lp)

    def _block(eb, sem, rowb, pre_rowb, do_pre, lp):
        pltpu.make_async_copy(edges.at[pl.ds(rowb, BROWS)], eb, sem).wait()
        for r in range(BROWS):
            for g in range(SUB // 16):
                sl = pl.ds(g * 16, 16)
                sv = eb[r, 0, sl]
                dv = eb[r, 1, sl]
                wv = eb[r, 2, sl]
                ok = (dv >= base_c) & (dv < base_c + HALF)
                cs = plsc.cumsum(jnp.where(ok, jnp.int32(1), jnp.int32(0)))
                pos = jnp.maximum(lp + cs - 1, 0)
                row = jax.lax.shift_right_logical(pos, 7)
                col = jax.lax.bitwise_and(pos, 127)
                plsc.store_scatter(csrc, [row, col], sv, mask=ok)
                plsc.store_scatter(cdst, [row, col], dv, mask=ok)
                plsc.store_scatter(cw, [row, col], wv, mask=ok)
                lp = lp + cs[15]

        @pl.when(do_pre)
        def _():
            pltpu.async_copy(edges.at[pl.ds(pre_rowb, BROWS)], eb, sem)
        return _flush(lp)

    def _pair(i, lp):
        rowbA = ibase + 2 * i * BROWS
        rowbB = rowbA + BROWS
        lp = _block(ebA, semA, rowbA, rowbA + 2 * BROWS, i + 1 < NBLK // 2, lp)
        lp = _block(ebB, semB, rowbB, rowbB + 2 * BROWS, i + 1 < NBLK // 2, lp)
        return lp
    lp = lax.fori_loop(0, NBLK // 2, _pair, jnp.int32(0))

    for r in range(BROWS):
        for g in range(SUB // 16):
            sl = pl.ds(g * 16, 16)
            pvec = lax.iota(jnp.int32, 16) + (r * SUB + g * 16)
            keep = pvec < lp
            zero = jnp.zeros((16,), jnp.int32)
            csrc[r, sl] = jnp.where(keep, csrc[r, sl], zero)
            cdst[r, sl] = jnp.where(keep, cdst[r, sl], zero)
            cw[r, sl] = jnp.where(keep, cw[r, sl], zero)

    @pl.when(lp > 0)
    def _():
        grow = plsc.fetch_and_add(cnt.at[0], jnp.int32(BROWS), subcore_id=0)

        @pl.when(grow <= PADR - BROWS)
        def _():
            pltpu.sync_copy(csrc.at[pl.ds(0, BROWS)],
                            srcp.at[c].at[pl.ds(grow, BROWS)])
            pltpu.sync_copy(cdst.at[pl.ds(0, BROWS)],
                            dstp.at[c].at[pl.ds(grow, BROWS)])
            pltpu.sync_copy(cw.at[pl.ds(0, BROWS)],
                            wp.at[c].at[pl.ds(grow, BROWS)])
    plsc.subcore_barrier()

    @pl.when(s == 0)
    def _():
        total = cnt[0]
        cbuf[pl.ds(0, 16)] = jnp.full((16,), total, jnp.int32)
        pltpu.sync_copy(cbuf, counts.at[c])


@functools.partial(
    pl.kernel,
    out_type=jax.ShapeDtypeStruct((NPAD, EMB), jnp.float32),
    mesh=_MESH,
    scratch_types=[
        pltpu.VMEM((3, 4, SUB), jnp.int32),
        pltpu.VMEM((3, 4, SUB), jnp.int32),
        pltpu.VMEM((3, 4, SUB), jnp.int32),
        pltpu.VMEM((3 * SUB, EMB), jnp.float32),
        pltpu.VMEM((16,), jnp.int32),
        pltpu.VMEM_SHARED((ACC_ROWS, EMB), jnp.float32),
        pltpu.SemaphoreType.DMA,
        pltpu.SemaphoreType.DMA,
        pltpu.SemaphoreType.DMA,
        pltpu.SemaphoreType.DMA,
        pltpu.SemaphoreType.DMA,
        pltpu.SemaphoreType.DMA,
        pltpu.SemaphoreType.DMA,
        pltpu.SemaphoreType.DMA,
        pltpu.SemaphoreType.DMA,
    ],
    compiler_params=_SC_PARAMS,
)
def _layer(ego, srcp, dstp, wp, counts, out,
           sb, db, wb, rows, cbuf, acc,
           semt0, semt1, semt2, semg0, semg1, semg2, sems0, sems1, sems2):
    c = lax.axis_index("c")
    s = lax.axis_index("s")
    base_c = c * HALF
    rbase = s * TROWS
    lbase = s * LTROWS
    semt = (semt0, semt1, semt2)
    semg = (semg0, semg1, semg2)
    sems = (sems0, sems1, sems2)

    pltpu.sync_copy(counts.at[c], cbuf)
    cnt_rows = cbuf[pl.ds(0, 16)][0]

    def act(t):
        return lbase + t < cnt_rows

    ZR = 3 * SUB

    def _zrow(i, carry):
        for k in range(EMB // 16):
            rows[i, pl.ds(k * 16, 16)] = jnp.zeros((16,), jnp.float32)
        return carry
    lax.fori_loop(0, ZR, _zrow, 0)
    for b in range(TROWS // ZR):
        pltpu.sync_copy(rows.at[pl.ds(0, ZR)],
                        acc.at[pl.ds(rbase + b * ZR, ZR)])
    rem = TROWS % ZR
    if rem:
        pltpu.sync_copy(rows.at[pl.ds(0, rem)],
                        acc.at[pl.ds(rbase + TROWS - rem, rem)])
    plsc.subcore_barrier()

    def _stage(slot, grow):
        pltpu.async_copy(srcp.at[c].at[pl.ds(grow, 4)], sb.at[slot],
                         semt[slot])
        pltpu.async_copy(dstp.at[c].at[pl.ds(grow, 4)], db.at[slot],
                         semt[slot])
        pltpu.async_copy(wp.at[c].at[pl.ds(grow, 4)], wb.at[slot],
                         semt[slot])

    def _wait_stage(slot, grow):
        pltpu.make_async_copy(srcp.at[c].at[pl.ds(grow, 4)], sb.at[slot],
                              semt[slot]).wait()
        pltpu.make_async_copy(dstp.at[c].at[pl.ds(grow, 4)], db.at[slot],
                              semt[slot]).wait()
        pltpu.make_async_copy(wp.at[c].at[pl.ds(grow, 4)], wb.at[slot],
                              semt[slot]).wait()

    def _fire_gather(u24, t):
        gslot = u24 % 3
        st_slot = (u24 // 4) % 3
        st_row = u24 % 4
        pltpu.async_copy(ego.at[sb.at[st_slot, st_row]],
                         rows.at[pl.ds(gslot * SUB, SUB)], semg[gslot])

    def _wait_gather(u24):
        gslot = u24 % 3
        st_slot = (u24 // 4) % 3
        st_row = u24 % 4
        pltpu.make_async_copy(ego.at[sb.at[st_slot, st_row]],
                              rows.at[pl.ds(gslot * SUB, SUB)],
                              semg[gslot]).wait()

    def _wait_scatter(u24):
        gslot = u24 % 3
        st_slot = (u24 // 4) % 3
        st_row = u24 % 4
        pltpu.make_async_copy(rows.at[pl.ds(gslot * SUB, SUB)],
                              acc.at[db.at[st_slot, st_row]],
                              sems[gslot]).wait()

    @pl.when(act(0))
    def _():
        _stage(0, lbase)

    @pl.when(act(4))
    def _():
        _stage(1, lbase + 4)

    @pl.when(act(0))
    def _():
        _wait_stage(0, lbase)
        _fire_gather(0, 0)

    @pl.when(act(1))
    def _():
        _fire_gather(1, 1)

    def _body(i, carry):
        tb = i * UPB
        for u in range(UPB):
            t = tb + u
            gslot = u % 3
            st_slot = (u // 4) % 3
            st_row = u % 4

            @pl.when(act(t))
            def _(u=u, t=t, gslot=gslot, st_slot=st_slot, st_row=st_row):
                _wait_gather(u)
                for k in range(SUB // 16):
                    v = db[st_slot, st_row, pl.ds(k * 16, 16)]
                    ok = (v >= base_c) & (v < base_c + HALF)
                    db[st_slot, st_row, pl.ds(k * 16, 16)] = jnp.where(
                        ok, v - base_c, DUMMY)

                @plsc.parallel_loop(0, SUB // 16, 1)
                def _scale(g):
                    wv = plsc.bitcast(wb[st_slot, st_row, pl.ds(g * 16, 16)],
                                      jnp.float32)
                    for i2 in range(16):
                        e = gslot * SUB + g * 16 + i2
                        w = wv[i2]
                        for k in range(EMB // 16):
                            rows[e, pl.ds(k * 16, 16)] = (
                                rows[e, pl.ds(k * 16, 16)] * w)

                pltpu.async_copy(rows.at[pl.ds(gslot * SUB, SUB)],
                                 acc.at[db.at[st_slot, st_row]],
                                 sems[gslot], add=True)

            if u % 4 == 2:
                @pl.when((t + 2 < LTROWS) & act(t + 2))
                def _(u=u, t=t):
                    _wait_stage(((u + 2) // 4) % 3, lbase + t + 2)

            @pl.when((t >= 1) & act(t - 1))
            def _(u=u):
                _wait_scatter((u - 1) % 24)

            @pl.when((t + 2 < LTROWS) & act(t + 2))
            def _(u=u, t=t):
                _fire_gather((u + 2) % 24, t + 2)

            if u % 4 == 0:
                @pl.when((t + 8 < LTROWS) & act(t + 8))
                def _(u=u, t=t):
                    _stage(((u // 4) + 2) % 3, lbase + t + 8)
        return carry
    lax.fori_loop(0, NBODY, _body, 0)

    @pl.when(act(LTROWS - 1))
    def _():
        _wait_scatter((LTROWS - 1) % 24)
    plsc.subcore_barrier()

    for b in range(TROWS // ZR):
        pltpu.sync_copy(acc.at[pl.ds(rbase + b * ZR, ZR)], rows)
        pltpu.sync_copy(rows, out.at[pl.ds(base_c + rbase + b * ZR, ZR)])
    if rem:
        pltpu.sync_copy(acc.at[pl.ds(rbase + TROWS - rem, rem)],
                        rows.at[pl.ds(0, rem)])
        pltpu.sync_copy(rows.at[pl.ds(0, rem)],
                        out.at[pl.ds(base_c + rbase + TROWS - rem, rem)])


def _mean_body(a_ref, b_ref, c_ref, o_ref):
    o_ref[...] = (a_ref[...] + b_ref[...] + c_ref[...]) * (1.0 / 3.0)


_MEAN_BLOCK = 2000


def _mean3(e1, e2, e3):
    spec = pl.BlockSpec((_MEAN_BLOCK, EMB), lambda i: (i, 0))
    return pl.pallas_call(
        _mean_body,
        grid=(NNODES // _MEAN_BLOCK,),
        in_specs=[spec, spec, spec],
        out_specs=spec,
        out_shape=jax.ShapeDtypeStruct((NNODES, EMB), jnp.float32),
    )(e1, e2, e3)


def kernel(user_emb, item_emb, edge_index, edge_weight):
    ego0 = jnp.concatenate([user_emb, item_emb], axis=0)
    ego0 = jnp.pad(ego0, ((0, NPAD - NNODES), (0, 0)))
    dst = edge_index[0].astype(jnp.int32)
    src = edge_index[1].astype(jnp.int32)
    w = edge_weight.astype(jnp.float32)
    src = jnp.pad(src, (0, EPAD - NEDGES))
    dst = jnp.pad(dst, (0, EPAD - NEDGES), constant_values=jnp.int32(2 ** 30))
    wbits = lax.bitcast_convert_type(jnp.pad(w, (0, EPAD - NEDGES)), jnp.int32)
    edges = jnp.stack([src.reshape(EROWS, SUB), dst.reshape(EROWS, SUB),
                       wbits.reshape(EROWS, SUB)], axis=1)

    srcp, dstp, wp, counts = _partition(edges)
    e1 = _layer(ego0, srcp, dstp, wp, counts)
    e2 = _layer(e1, srcp, dstp, wp, counts)
    e3 = _layer(e2, srcp, dstp, wp, counts)
    fin = _mean3(e1, e2, e3)
    return fin[:USERS], fin[USERS:]
